# baseline (speedup 1.0000x reference)
import functools
import numpy as np
import jax
import jax.numpy as jnp
from jax import lax
from jax.experimental import pallas as pl
from jax.experimental.pallas import tpu as pltpu
from jax.experimental.pallas import tpu_sc as plsc

_N = 10000
_E = 160000
_MUL0 = 16
_MUL1 = 8
_DIM = 40
_NUM_BASIS = 16
_HIDDEN = 64
_WN = 576

_NC = 2
_NS = 16
_NW = _NC * _NS
_CHUNK = 128
_NCH = 40
_EPAD = _NW * _NCH * _CHUNK
_GROW = 48
_GCOL = 16
_NACC = 10240
_RPT = _NACC // _NS

_BE = 1024


def _sc_gather_body(trow_hbm, tcol_hbm, idxr_hbm, idxc_hbm, grow_hbm, gcol_hbm,
                    idxr_v, idxc_v, bufr_v, bufc_v, semr, semc):
    cid = lax.axis_index("c")
    sid = lax.axis_index("s")
    wid = sid * _NC + cid
    pltpu.sync_copy(idxr_hbm.at[wid], idxr_v)
    pltpu.sync_copy(idxc_hbm.at[wid], idxc_v)

    def chunk(j, carry):
        base = (wid * _NCH + j) * _CHUNK
        cpr = pltpu.async_copy(trow_hbm.at[idxr_v.at[j]], bufr_v, semr)
        cpc = pltpu.async_copy(tcol_hbm.at[idxc_v.at[j]], bufc_v, semc)
        cpr.wait()
        pltpu.sync_copy(bufr_v, grow_hbm.at[pl.ds(base, _CHUNK)])
        cpc.wait()
        pltpu.sync_copy(bufc_v, gcol_hbm.at[pl.ds(base, _CHUNK)])
        return carry

    lax.fori_loop(0, _NCH, chunk, 0)


def _sc_gather(trow, tcol, idxr, idxc):
    mesh = plsc.VectorSubcoreMesh(core_axis_name="c", subcore_axis_name="s",
                                  num_cores=_NC, num_subcores=_NS)
    return pl.kernel(
        _sc_gather_body,
        out_type=[
            jax.ShapeDtypeStruct((_EPAD, _GROW), jnp.float32),
            jax.ShapeDtypeStruct((_EPAD, _GCOL), jnp.float32),
        ],
        mesh=mesh,
        scratch_types=[
            pltpu.VMEM((_NCH, _CHUNK), jnp.int32),
            pltpu.VMEM((_NCH, _CHUNK), jnp.int32),
            pltpu.VMEM((_CHUNK, _GROW), jnp.float32),
            pltpu.VMEM((_CHUNK, _GCOL), jnp.float32),
            pltpu.SemaphoreType.DMA,
            pltpu.SemaphoreType.DMA,
        ],
        compiler_params=pltpu.CompilerParams(use_tc_tiling_on_sc=False),
    )(trow, tcol, idxr, idxc)


def _tc_msg_body(grow_ref, gcol_ref, w1_ref, w2_ref, out_ref):
    g = grow_ref[...]
    xs = g[:, 0:16]
    ev = g[:, 40:43] - gcol_ref[:, 0:3]
    elen = jnp.sqrt(jnp.sum(ev * ev, axis=1, keepdims=True))
    centers = lax.broadcasted_iota(jnp.int32, (1, _NUM_BASIS), 1).astype(
        jnp.float32) * np.float32(4.0 / (_NUM_BASIS - 1))
    rbf = jnp.exp(-8.0 * (elen - centers) ** 2)
    h = jnp.maximum(
        jnp.dot(rbf, w1_ref[...], preferred_element_type=jnp.float32), 0.0)
    w = jnp.dot(h, w2_ref[...], preferred_element_type=jnp.float32)

    unit = ev / jnp.maximum(elen, 1e-12)
    y1 = jnp.sqrt(3.0).astype(jnp.float32) * unit
    xvY = (g[:, 16:24] * y1[:, 0:1] + g[:, 24:32] * y1[:, 1:2]
           + g[:, 32:40] * y1[:, 2:3])

    a0 = w[:, 0:16] * xs[:, 0:1]
    for u in range(1, 16):
        a0 = a0 + w[:, u * 16:(u + 1) * 16] * xs[:, u:u + 1]
    b0 = w[:, 256:272] * xvY[:, 0:1]
    for u in range(1, 8):
        b0 = b0 + w[:, 256 + u * 16:256 + (u + 1) * 16] * xvY[:, u:u + 1]
    out0 = a0 + b0

    c1 = w[:, 384:392] * xs[:, 0:1]
    for u in range(1, 16):
        c1 = c1 + w[:, 384 + u * 8:384 + (u + 1) * 8] * xs[:, u:u + 1]

    outs = [out0]
    for k in range(3):
        d1 = w[:, 512:520] * g[:, 16 + 8 * k:17 + 8 * k]
        for u in range(1, 8):
            d1 = d1 + w[:, 512 + u * 8:512 + (u + 1) * 8] \
                * g[:, 16 + 8 * k + u:17 + 8 * k + u]
        outs.append(c1 * y1[:, k:k + 1] + d1)
    outs.append(jnp.zeros((g.shape[0], 8), jnp.float32))
    out_ref[...] = jnp.concatenate(outs, axis=1)


def _tc_msg(grow, gcol, w1p, w2p):
    grid = (_EPAD // _BE,)
    return pl.pallas_call(
        _tc_msg_body,
        grid=grid,
        in_specs=[
            pl.BlockSpec((_BE, _GROW), lambda i: (i, 0)),
            pl.BlockSpec((_BE, _GCOL), lambda i: (i, 0)),
            pl.BlockSpec((_NUM_BASIS, _HIDDEN), lambda i: (0, 0)),
            pl.BlockSpec((_HIDDEN, _WN), lambda i: (0, 0)),
        ],
        out_specs=pl.BlockSpec((_BE, _GROW), lambda i: (i, 0)),
        out_shape=jax.ShapeDtypeStruct((_EPAD, _GROW), jnp.float32),
    )(grow, gcol, w1p, w2p)


def _sc_scatter_body(msg_hbm, idxd_hbm, zeros_hbm, out_hbm,
                     acc_shared, idx_v, buf_v):
    cid = lax.axis_index("c")
    sid = lax.axis_index("s")
    wid = sid * _NC + cid
    pltpu.sync_copy(zeros_hbm, acc_shared.at[pl.ds(sid * _RPT, _RPT)])
    pltpu.sync_copy(idxd_hbm.at[wid], idx_v)
    plsc.subcore_barrier()

    def chunk(j, carry):
        base = (wid * _NCH + j) * _CHUNK
        pltpu.sync_copy(msg_hbm.at[pl.ds(base, _CHUNK)], buf_v)
        pltpu.sync_copy(buf_v, acc_shared.at[idx_v.at[j]], add=True)
        return carry

    lax.fori_loop(0, _NCH, chunk, 0)
    plsc.subcore_barrier()
    pltpu.sync_copy(acc_shared.at[pl.ds(sid * _RPT, _RPT)],
                    out_hbm.at[cid].at[pl.ds(sid * _RPT, _RPT)])


def _sc_scatter(msg, idxd, zeros_blk):
    mesh = plsc.VectorSubcoreMesh(core_axis_name="c", subcore_axis_name="s",
                                  num_cores=_NC, num_subcores=_NS)
    return pl.kernel(
        _sc_scatter_body,
        out_type=jax.ShapeDtypeStruct((_NC, _NACC, _GROW), jnp.float32),
        mesh=mesh,
        scratch_types=[
            pltpu.VMEM_SHARED((_NACC, _GROW), jnp.float32),
            pltpu.VMEM((_NCH, _CHUNK), jnp.int32),
            pltpu.VMEM((_CHUNK, _GROW), jnp.float32),
        ],
        compiler_params=pltpu.CompilerParams(use_tc_tiling_on_sc=False),
    )(msg, idxd, zeros_blk)


def _tc_fin_body(p_ref, out_ref):
    s = p_ref[0] + p_ref[1]
    sc = s[:, 0:16]
    act = sc / (1.0 + jnp.exp(-sc))
    out_ref[...] = jnp.concatenate([act, s[:, 16:48]], axis=1)


def _tc_fin(partials):
    return pl.pallas_call(
        _tc_fin_body,
        out_shape=jax.ShapeDtypeStruct((_NACC, _GROW), jnp.float32),
    )(partials)


def kernel(x, pos, edge_index, W1, W2):
    f32 = jnp.float32
    row = edge_index[0].astype(jnp.int32)
    col = edge_index[1].astype(jnp.int32)

    pw0 = np.sqrt(1.0 / 24.0)
    pw1 = np.sqrt(3.0 / 24.0)
    inv_s3 = 1.0 / np.sqrt(3.0)
    colscale = np.concatenate([
        np.full(256, pw0), np.full(128, pw0 * inv_s3),
        np.full(128, pw1 * inv_s3), np.full(64, pw1 * inv_s3),
    ]).astype(np.float32)
    w1p = W1 * np.float32(np.sqrt(2.0) / np.sqrt(_NUM_BASIS))
    w2p = (W2 * np.float32(1.0 / np.sqrt(_HIDDEN))) * colscale[None, :]

    xk = jnp.concatenate(
        [x[:, :16],
         x[:, 16:].reshape(_N, 8, 3).transpose(0, 2, 1).reshape(_N, 24)],
        axis=1)
    trow = jnp.concatenate([xk, pos, jnp.zeros((_N, 5), f32)], axis=1)
    tcol = jnp.concatenate([pos, jnp.zeros((_N, 13), f32)], axis=1)

    pad = _EPAD - _E
    idxr = jnp.pad(row, (0, pad)).reshape(_NW, _NCH, _CHUNK)
    idxc = jnp.pad(col, (0, pad)).reshape(_NW, _NCH, _CHUNK)
    idxd = jnp.pad(row, (0, pad), constant_values=_NACC - 1).reshape(
        _NW, _NCH, _CHUNK)
    zeros_blk = jnp.zeros((_RPT, _GROW), f32)

    grow, gcol = _sc_gather(trow, tcol, idxr, idxc)
    msg = _tc_msg(grow, gcol, w1p, w2p)
    partials = _sc_scatter(msg, idxd, zeros_blk)
    yfull = _tc_fin(partials)

    ys = yfull[:_N, 0:16]
    yv = yfull[:_N, 16:40].reshape(_N, 3, 8).transpose(0, 2, 1).reshape(_N, 24)
    return jnp.concatenate([ys, yv], axis=1)


# speedup vs baseline: 3.7209x; 3.7209x over previous
import functools
import numpy as np
import jax
import jax.numpy as jnp
from jax import lax
from jax.experimental import pallas as pl
from jax.experimental.pallas import tpu as pltpu
from jax.experimental.pallas import tpu_sc as plsc

_N = 10000
_E = 160000
_MUL0 = 16
_MUL1 = 8
_DIM = 40
_NUM_BASIS = 16
_HIDDEN = 64
_WN = 576

_NC = 2
_NS = 16
_NW = _NC * _NS
_CHUNK = 128
_NCH = 40
_EPAD = _NW * _NCH * _CHUNK
_GROW = 48
_GCOL = 16
_NACC = 10240
_RPT = _NACC // _NS

_BE = 1024


def _sc_gather_body(trow_hbm, tcol_hbm, idxr_hbm, idxc_hbm, grow_hbm, gcol_hbm,
                    idxr_v, idxc_v, bufr_v, bufc_v, semr, semc):
    cid = lax.axis_index("c")
    sid = lax.axis_index("s")
    wid = sid * _NC + cid
    pltpu.sync_copy(idxr_hbm.at[wid], idxr_v)
    pltpu.sync_copy(idxc_hbm.at[wid], idxc_v)

    def chunk(j, carry):
        base = (wid * _NCH + j) * _CHUNK
        cpr = pltpu.async_copy(trow_hbm.at[idxr_v.at[j]], bufr_v, semr)
        cpc = pltpu.async_copy(tcol_hbm.at[idxc_v.at[j]], bufc_v, semc)
        cpr.wait()
        pltpu.sync_copy(bufr_v, grow_hbm.at[pl.ds(base, _CHUNK)])
        cpc.wait()
        pltpu.sync_copy(bufc_v, gcol_hbm.at[pl.ds(base, _CHUNK)])
        return carry

    lax.fori_loop(0, _NCH, chunk, 0)


def _sc_gather(trow, tcol, idxr, idxc):
    mesh = plsc.VectorSubcoreMesh(core_axis_name="c", subcore_axis_name="s",
                                  num_cores=_NC, num_subcores=_NS)
    return pl.kernel(
        _sc_gather_body,
        out_type=[
            jax.ShapeDtypeStruct((_EPAD, _GROW), jnp.float32),
            jax.ShapeDtypeStruct((_EPAD, _GCOL), jnp.float32),
        ],
        mesh=mesh,
        scratch_types=[
            pltpu.VMEM((_NCH, _CHUNK), jnp.int32),
            pltpu.VMEM((_NCH, _CHUNK), jnp.int32),
            pltpu.VMEM((_CHUNK, _GROW), jnp.float32),
            pltpu.VMEM((_CHUNK, _GCOL), jnp.float32),
            pltpu.SemaphoreType.DMA,
            pltpu.SemaphoreType.DMA,
        ],
        compiler_params=pltpu.CompilerParams(use_tc_tiling_on_sc=False),
    )(trow, tcol, idxr, idxc)


def _dot(a, b):
    return jnp.dot(a, b, preferred_element_type=jnp.float32)


def _tc_msg_body(grow_ref, gcol_ref, w1_ref, w2_ref, r1_ref, s1_ref, r2_ref,
                 s2_ref, r3_ref, s3_ref, out_ref):
    g = grow_ref[...]
    xs = g[:, 0:16]
    ev = g[:, 40:43] - gcol_ref[:, 0:3]
    elen = jnp.sqrt(jnp.sum(ev * ev, axis=1, keepdims=True))
    centers = lax.broadcasted_iota(jnp.int32, (1, _NUM_BASIS), 1).astype(
        jnp.float32) * np.float32(4.0 / (_NUM_BASIS - 1))
    rbf = jnp.exp(-8.0 * (elen - centers) ** 2)
    h = jnp.maximum(_dot(rbf, w1_ref[...]), 0.0)
    w = _dot(h, w2_ref[...])

    unit = ev / jnp.maximum(elen, 1e-12)
    y1 = jnp.sqrt(3.0).astype(jnp.float32) * unit
    xvY = (g[:, 16:24] * y1[:, 0:1] + g[:, 24:32] * y1[:, 1:2]
           + g[:, 32:40] * y1[:, 2:3])

    t = jnp.concatenate([xs, xvY], axis=1)
    out0 = _dot(w[:, 0:384] * _dot(t, r1_ref[...]), s1_ref[...])
    c1 = _dot(w[:, 384:512] * _dot(xs, r2_ref[...]), s2_ref[...])
    wd = w[:, 512:576]
    outs = [out0]
    for k in range(3):
        repd = _dot(g[:, 16 + 8 * k:24 + 8 * k], r3_ref[...])
        d1 = _dot(wd * repd, s3_ref[...])
        outs.append(c1 * y1[:, k:k + 1] + d1)
    outs.append(jnp.zeros((g.shape[0], 8), jnp.float32))
    out_ref[...] = jnp.concatenate(outs, axis=1)


def _tp_consts():
    r1 = np.zeros((24, 384), np.float32)
    for j in range(256):
        r1[j // 16, j] = 1.0
    for j in range(128):
        r1[16 + j // 16, 256 + j] = 1.0
    s1 = np.zeros((384, 16), np.float32)
    for j in range(384):
        s1[j, j % 16] = 1.0
    r2 = np.zeros((16, 128), np.float32)
    for j in range(128):
        r2[j // 8, j] = 1.0
    s2 = np.zeros((128, 8), np.float32)
    for j in range(128):
        s2[j, j % 8] = 1.0
    r3 = np.zeros((8, 64), np.float32)
    for j in range(64):
        r3[j // 8, j] = 1.0
    s3 = np.zeros((64, 8), np.float32)
    for j in range(64):
        s3[j, j % 8] = 1.0
    return (jnp.asarray(r1), jnp.asarray(s1), jnp.asarray(r2),
            jnp.asarray(s2), jnp.asarray(r3), jnp.asarray(s3))


def _tc_msg(grow, gcol, w1p, w2p):
    grid = (_EPAD // _BE,)
    consts = _tp_consts()
    full = lambda a: pl.BlockSpec(a.shape, lambda i: (0,) * a.ndim)
    return pl.pallas_call(
        _tc_msg_body,
        grid=grid,
        in_specs=[
            pl.BlockSpec((_BE, _GROW), lambda i: (i, 0)),
            pl.BlockSpec((_BE, _GCOL), lambda i: (i, 0)),
            pl.BlockSpec((_NUM_BASIS, _HIDDEN), lambda i: (0, 0)),
            pl.BlockSpec((_HIDDEN, _WN), lambda i: (0, 0)),
        ] + [full(c) for c in consts],
        out_specs=pl.BlockSpec((_BE, _GROW), lambda i: (i, 0)),
        out_shape=jax.ShapeDtypeStruct((_EPAD, _GROW), jnp.float32),
    )(grow, gcol, w1p, w2p, *consts)


def _sc_scatter_body(msg_hbm, idxd_hbm, zeros_hbm, out_hbm,
                     acc_shared, idx_v, buf_v):
    cid = lax.axis_index("c")
    sid = lax.axis_index("s")
    wid = sid * _NC + cid
    pltpu.sync_copy(zeros_hbm, acc_shared.at[pl.ds(sid * _RPT, _RPT)])
    pltpu.sync_copy(idxd_hbm.at[wid], idx_v)
    plsc.subcore_barrier()

    def chunk(j, carry):
        base = (wid * _NCH + j) * _CHUNK
        pltpu.sync_copy(msg_hbm.at[pl.ds(base, _CHUNK)], buf_v)
        pltpu.sync_copy(buf_v, acc_shared.at[idx_v.at[j]], add=True)
        return carry

    lax.fori_loop(0, _NCH, chunk, 0)
    plsc.subcore_barrier()
    pltpu.sync_copy(acc_shared.at[pl.ds(sid * _RPT, _RPT)],
                    out_hbm.at[cid].at[pl.ds(sid * _RPT, _RPT)])


def _sc_scatter(msg, idxd, zeros_blk):
    mesh = plsc.VectorSubcoreMesh(core_axis_name="c", subcore_axis_name="s",
                                  num_cores=_NC, num_subcores=_NS)
    return pl.kernel(
        _sc_scatter_body,
        out_type=jax.ShapeDtypeStruct((_NC, _NACC, _GROW), jnp.float32),
        mesh=mesh,
        scratch_types=[
            pltpu.VMEM_SHARED((_NACC, _GROW), jnp.float32),
            pltpu.VMEM((_NCH, _CHUNK), jnp.int32),
            pltpu.VMEM((_CHUNK, _GROW), jnp.float32),
        ],
        compiler_params=pltpu.CompilerParams(use_tc_tiling_on_sc=False),
    )(msg, idxd, zeros_blk)


def _tc_fin_body(p_ref, out_ref):
    s = p_ref[0] + p_ref[1]
    sc = s[:, 0:16]
    act = sc / (1.0 + jnp.exp(-sc))
    out_ref[...] = jnp.concatenate([act, s[:, 16:48]], axis=1)


def _tc_fin(partials):
    return pl.pallas_call(
        _tc_fin_body,
        out_shape=jax.ShapeDtypeStruct((_NACC, _GROW), jnp.float32),
    )(partials)


def kernel(x, pos, edge_index, W1, W2):
    f32 = jnp.float32
    row = edge_index[0].astype(jnp.int32)
    col = edge_index[1].astype(jnp.int32)

    pw0 = np.sqrt(1.0 / 24.0)
    pw1 = np.sqrt(3.0 / 24.0)
    inv_s3 = 1.0 / np.sqrt(3.0)
    colscale = np.concatenate([
        np.full(256, pw0), np.full(128, pw0 * inv_s3),
        np.full(128, pw1 * inv_s3), np.full(64, pw1 * inv_s3),
    ]).astype(np.float32)
    w1p = W1 * np.float32(np.sqrt(2.0) / np.sqrt(_NUM_BASIS))
    w2p = (W2 * np.float32(1.0 / np.sqrt(_HIDDEN))) * colscale[None, :]

    xk = jnp.concatenate(
        [x[:, :16],
         x[:, 16:].reshape(_N, 8, 3).transpose(0, 2, 1).reshape(_N, 24)],
        axis=1)
    trow = jnp.concatenate([xk, pos, jnp.zeros((_N, 5), f32)], axis=1)
    tcol = jnp.concatenate([pos, jnp.zeros((_N, 13), f32)], axis=1)

    pad = _EPAD - _E
    idxr = jnp.pad(row, (0, pad)).reshape(_NW, _NCH, _CHUNK)
    idxc = jnp.pad(col, (0, pad)).reshape(_NW, _NCH, _CHUNK)
    idxd = jnp.pad(row, (0, pad), constant_values=_NACC - 1).reshape(
        _NW, _NCH, _CHUNK)
    zeros_blk = jnp.zeros((_RPT, _GROW), f32)

    grow, gcol = _sc_gather(trow, tcol, idxr, idxc)
    msg = _tc_msg(grow, gcol, w1p, w2p)
    partials = _sc_scatter(msg, idxd, zeros_blk)
    yfull = _tc_fin(partials)

    ys = yfull[:_N, 0:16]
    yv = yfull[:_N, 16:40].reshape(_N, 3, 8).transpose(0, 2, 1).reshape(_N, 24)
    return jnp.concatenate([ys, yv], axis=1)


# speedup vs baseline: 4.0169x; 1.0796x over previous
import functools
import numpy as np
import jax
import jax.numpy as jnp
from jax import lax
from jax.experimental import pallas as pl
from jax.experimental.pallas import tpu as pltpu
from jax.experimental.pallas import tpu_sc as plsc

_N = 10000
_E = 160000
_MUL0 = 16
_MUL1 = 8
_DIM = 40
_NUM_BASIS = 16
_HIDDEN = 64
_WN = 576

_NC = 2
_NS = 16
_NW = _NC * _NS
_CHUNK = 128
_NCH = 40
_EPAD = _NW * _NCH * _CHUNK
_GROW = 48
_GCOL = 16
_NACC = 10240
_RPT = _NACC // _NS

_BE = 2048


def _sc_gather_body(trow_hbm, tcol_hbm, idxr_hbm, idxc_hbm, grow_hbm, gcol_hbm,
                    idxr_v, idxc_v, bufr_v, bufc_v, semr, semc, semo):
    cid = lax.axis_index("c")
    sid = lax.axis_index("s")
    wid = sid * _NC + cid
    pltpu.sync_copy(idxr_hbm.at[wid], idxr_v)
    pltpu.sync_copy(idxc_hbm.at[wid], idxc_v)
    pltpu.async_copy(trow_hbm.at[idxr_v.at[0]], bufr_v.at[0], semr)
    pltpu.async_copy(tcol_hbm.at[idxc_v.at[0]], bufc_v.at[0], semc)

    def chunk(j, carry):
        cur = lax.rem(j, 2)
        nxt = lax.rem(j + 1, 2)
        base = (wid * _NCH + j) * _CHUNK
        pltpu.make_async_copy(trow_hbm.at[idxr_v.at[j]],
                              bufr_v.at[cur], semr).wait()
        pltpu.make_async_copy(tcol_hbm.at[idxc_v.at[j]],
                              bufc_v.at[cur], semc).wait()

        @pl.when(j > 0)
        def _():
            pbase = (wid * _NCH + j - 1) * _CHUNK
            pltpu.make_async_copy(bufr_v.at[nxt],
                                  grow_hbm.at[pl.ds(pbase, _CHUNK)],
                                  semo).wait()
            pltpu.make_async_copy(bufc_v.at[nxt],
                                  gcol_hbm.at[pl.ds(pbase, _CHUNK)],
                                  semo).wait()

        @pl.when(j + 1 < _NCH)
        def _():
            pltpu.async_copy(trow_hbm.at[idxr_v.at[j + 1]],
                             bufr_v.at[nxt], semr)
            pltpu.async_copy(tcol_hbm.at[idxc_v.at[j + 1]],
                             bufc_v.at[nxt], semc)

        pltpu.async_copy(bufr_v.at[cur], grow_hbm.at[pl.ds(base, _CHUNK)],
                         semo)
        pltpu.async_copy(bufc_v.at[cur], gcol_hbm.at[pl.ds(base, _CHUNK)],
                         semo)
        return carry

    lax.fori_loop(0, _NCH, chunk, 0)
    lbase = (wid * _NCH + _NCH - 1) * _CHUNK
    lpar = (_NCH - 1) % 2
    pltpu.make_async_copy(bufr_v.at[lpar],
                          grow_hbm.at[pl.ds(lbase, _CHUNK)], semo).wait()
    pltpu.make_async_copy(bufc_v.at[lpar],
                          gcol_hbm.at[pl.ds(lbase, _CHUNK)], semo).wait()


def _sc_gather(trow, tcol, idxr, idxc):
    mesh = plsc.VectorSubcoreMesh(core_axis_name="c", subcore_axis_name="s",
                                  num_cores=_NC, num_subcores=_NS)
    return pl.kernel(
        _sc_gather_body,
        out_type=[
            jax.ShapeDtypeStruct((_EPAD, _GROW), jnp.float32),
            jax.ShapeDtypeStruct((_EPAD, _GCOL), jnp.float32),
        ],
        mesh=mesh,
        scratch_types=[
            pltpu.VMEM((_NCH, _CHUNK), jnp.int32),
            pltpu.VMEM((_NCH, _CHUNK), jnp.int32),
            pltpu.VMEM((2, _CHUNK, _GROW), jnp.float32),
            pltpu.VMEM((2, _CHUNK, _GCOL), jnp.float32),
            pltpu.SemaphoreType.DMA,
            pltpu.SemaphoreType.DMA,
            pltpu.SemaphoreType.DMA,
        ],
        compiler_params=pltpu.CompilerParams(use_tc_tiling_on_sc=False),
    )(trow, tcol, idxr, idxc)


def _dot(a, b):
    return jnp.dot(a, b, preferred_element_type=jnp.float32)


def _tc_msg_body(grow_ref, gcol_ref, w1_ref, w2_ref, r1_ref, s1_ref, r2_ref,
                 s2_ref, r3_ref, s3_ref, r4_ref, s4_ref, t4_ref, o3_ref,
                 o16_ref, out_ref):
    g = grow_ref[...]
    xs = g[:, 0:16]
    xv = g[:, 16:40]
    ev = g[:, 40:43] - gcol_ref[:, 0:3]
    len2 = _dot(ev * ev, o3_ref[...])
    elen = jnp.sqrt(len2)
    d16 = _dot(elen, o16_ref[...])
    centers = lax.broadcasted_iota(jnp.int32, (1, _NUM_BASIS), 1).astype(
        jnp.float32) * np.float32(4.0 / (_NUM_BASIS - 1))
    rbf = jnp.exp(-8.0 * (d16 - centers) ** 2)
    h = jnp.maximum(_dot(rbf, w1_ref[...]), 0.0)
    w = _dot(h, w2_ref[...])

    rlen = jnp.maximum(elen, 1e-12)
    y1 = ev * (jnp.sqrt(3.0).astype(jnp.float32) / rlen)
    xvY = (xv[:, 0:8] * y1[:, 0:1] + xv[:, 8:16] * y1[:, 1:2]
           + xv[:, 16:24] * y1[:, 2:3])

    t = jnp.concatenate([xs, xvY], axis=1)
    out0 = _dot(w[:, 0:384] * _dot(t, r1_ref[...]), s1_ref[...])
    c1 = _dot(w[:, 384:512] * _dot(xs, r2_ref[...]), s2_ref[...])
    wd = w[:, 512:576]
    outs = [out0]
    for k in range(3):
        repd = _dot(g[:, 16 + 8 * k:24 + 8 * k], r3_ref[...])
        d1 = _dot(wd * repd, s3_ref[...])
        outs.append(c1 * y1[:, k:k + 1] + d1)
    outs.append(jnp.zeros((g.shape[0], 8), jnp.float32))
    out_ref[...] = jnp.concatenate(outs, axis=1)


def _tp_consts():
    r1 = np.zeros((24, 384), np.float32)
    for j in range(256):
        r1[j // 16, j] = 1.0
    for j in range(128):
        r1[16 + j // 16, 256 + j] = 1.0
    s1 = np.zeros((384, 16), np.float32)
    for j in range(384):
        s1[j, j % 16] = 1.0
    r2 = np.zeros((16, 128), np.float32)
    for j in range(128):
        r2[j // 8, j] = 1.0
    s2 = np.zeros((128, 8), np.float32)
    for j in range(128):
        s2[j, j % 8] = 1.0
    r3 = np.zeros((8, 64), np.float32)
    for j in range(64):
        r3[j // 8, j] = 1.0
    s3 = np.zeros((64, 8), np.float32)
    for j in range(64):
        s3[j, j % 8] = 1.0
    r4 = np.zeros((3, 24), np.float32)
    for j in range(24):
        r4[j // 8, j] = 1.0
    s4 = np.zeros((24, 8), np.float32)
    for j in range(24):
        s4[j, j % 8] = 1.0
    t4 = np.zeros((8, 24), np.float32)
    for j in range(24):
        t4[j % 8, j] = 1.0
    o3 = np.ones((3, 1), np.float32)
    o16 = np.ones((1, 16), np.float32)
    mats = (r1, s1, r2, s2, r3, s3, r4, s4, t4, o3, o16)
    return tuple(jnp.asarray(m) for m in mats)


def _tc_msg(grow, gcol, w1p, w2p):
    grid = (_EPAD // _BE,)
    consts = _tp_consts()
    full = lambda a: pl.BlockSpec(a.shape, lambda i: (0,) * a.ndim)
    return pl.pallas_call(
        _tc_msg_body,
        grid=grid,
        in_specs=[
            pl.BlockSpec((_BE, _GROW), lambda i: (i, 0)),
            pl.BlockSpec((_BE, _GCOL), lambda i: (i, 0)),
            pl.BlockSpec((_NUM_BASIS, _HIDDEN), lambda i: (0, 0)),
            pl.BlockSpec((_HIDDEN, _WN), lambda i: (0, 0)),
        ] + [full(c) for c in consts],
        out_specs=pl.BlockSpec((_BE, _GROW), lambda i: (i, 0)),
        out_shape=jax.ShapeDtypeStruct((_EPAD, _GROW), jnp.float32),
    )(grow, gcol, w1p, w2p, *consts)


def _sc_scatter_body(msg_hbm, idxd_hbm, zeros_hbm, out_hbm,
                     acc_shared, idx_v, buf_v, seml):
    cid = lax.axis_index("c")
    sid = lax.axis_index("s")
    wid = sid * _NC + cid
    pltpu.sync_copy(zeros_hbm, acc_shared.at[pl.ds(sid * _RPT, _RPT)])
    pltpu.sync_copy(idxd_hbm.at[wid], idx_v)
    pltpu.async_copy(msg_hbm.at[pl.ds(wid * _NCH * _CHUNK, _CHUNK)],
                     buf_v.at[0], seml)
    plsc.subcore_barrier()

    def chunk(j, carry):
        cur = lax.rem(j, 2)
        base = (wid * _NCH + j) * _CHUNK
        pltpu.make_async_copy(msg_hbm.at[pl.ds(base, _CHUNK)],
                              buf_v.at[cur], seml).wait()

        @pl.when(j + 1 < _NCH)
        def _():
            nbase = (wid * _NCH + j + 1) * _CHUNK
            pltpu.async_copy(msg_hbm.at[pl.ds(nbase, _CHUNK)],
                             buf_v.at[lax.rem(j + 1, 2)], seml)

        pltpu.sync_copy(buf_v.at[cur], acc_shared.at[idx_v.at[j]], add=True)
        return carry

    lax.fori_loop(0, _NCH, chunk, 0)
    plsc.subcore_barrier()
    pltpu.sync_copy(acc_shared.at[pl.ds(sid * _RPT, _RPT)],
                    out_hbm.at[cid].at[pl.ds(sid * _RPT, _RPT)])


def _sc_scatter(msg, idxd, zeros_blk):
    mesh = plsc.VectorSubcoreMesh(core_axis_name="c", subcore_axis_name="s",
                                  num_cores=_NC, num_subcores=_NS)
    return pl.kernel(
        _sc_scatter_body,
        out_type=jax.ShapeDtypeStruct((_NC, _NACC, _GROW), jnp.float32),
        mesh=mesh,
        scratch_types=[
            pltpu.VMEM_SHARED((_NACC, _GROW), jnp.float32),
            pltpu.VMEM((_NCH, _CHUNK), jnp.int32),
            pltpu.VMEM((2, _CHUNK, _GROW), jnp.float32),
            pltpu.SemaphoreType.DMA,
        ],
        compiler_params=pltpu.CompilerParams(use_tc_tiling_on_sc=False),
    )(msg, idxd, zeros_blk)


def _tc_fin_body(p_ref, out_ref):
    s = p_ref[0] + p_ref[1]
    sc = s[:, 0:16]
    act = sc / (1.0 + jnp.exp(-sc))
    out_ref[...] = jnp.concatenate([act, s[:, 16:48]], axis=1)


def _tc_fin(partials):
    return pl.pallas_call(
        _tc_fin_body,
        out_shape=jax.ShapeDtypeStruct((_NACC, _GROW), jnp.float32),
    )(partials)


def kernel(x, pos, edge_index, W1, W2):
    f32 = jnp.float32
    row = edge_index[0].astype(jnp.int32)
    col = edge_index[1].astype(jnp.int32)

    pw0 = np.sqrt(1.0 / 24.0)
    pw1 = np.sqrt(3.0 / 24.0)
    inv_s3 = 1.0 / np.sqrt(3.0)
    colscale = np.concatenate([
        np.full(256, pw0), np.full(128, pw0 * inv_s3),
        np.full(128, pw1 * inv_s3), np.full(64, pw1 * inv_s3),
    ]).astype(np.float32)
    w1p = W1 * np.float32(np.sqrt(2.0) / np.sqrt(_NUM_BASIS))
    w2p = (W2 * np.float32(1.0 / np.sqrt(_HIDDEN))) * colscale[None, :]

    xk = jnp.concatenate(
        [x[:, :16],
         x[:, 16:].reshape(_N, 8, 3).transpose(0, 2, 1).reshape(_N, 24)],
        axis=1)
    trow = jnp.concatenate([xk, pos, jnp.zeros((_N, 5), f32)], axis=1)
    tcol = jnp.concatenate([pos, jnp.zeros((_N, 13), f32)], axis=1)

    pad = _EPAD - _E
    idxr = jnp.pad(row, (0, pad)).reshape(_NW, _NCH, _CHUNK)
    idxc = jnp.pad(col, (0, pad)).reshape(_NW, _NCH, _CHUNK)
    idxd = jnp.pad(row, (0, pad), constant_values=_NACC - 1).reshape(
        _NW, _NCH, _CHUNK)
    zeros_blk = jnp.zeros((_RPT, _GROW), f32)

    grow, gcol = _sc_gather(trow, tcol, idxr, idxc)
    msg = _tc_msg(grow, gcol, w1p, w2p)
    partials = _sc_scatter(msg, idxd, zeros_blk)
    yfull = _tc_fin(partials)

    ys = yfull[:_N, 0:16]
    yv = yfull[:_N, 16:40].reshape(_N, 3, 8).transpose(0, 2, 1).reshape(_N, 24)
    return jnp.concatenate([ys, yv], axis=1)


# speedup vs baseline: 4.9433x; 1.2306x over previous
import functools
import numpy as np
import jax
import jax.numpy as jnp
from jax import lax
from jax.experimental import pallas as pl
from jax.experimental.pallas import tpu as pltpu
from jax.experimental.pallas import tpu_sc as plsc

_N = 10000
_E = 160000
_MUL0 = 16
_MUL1 = 8
_DIM = 40
_NUM_BASIS = 16
_HIDDEN = 64
_WN = 576

_NC = 2
_NS = 16
_NW = _NC * _NS
_CHUNK = 128
_NCH = 40
_EPAD = _NW * _NCH * _CHUNK
_EPAD2 = _EPAD // 2
_GROW = 48
_GCOL = 16
_GW = 64
_NACC = 10240
_RPT = _NACC // _NS

_BE2 = 1024


def _sc_gather_body(trow_hbm, tcol_hbm, idxr_hbm, idxc_hbm, grow_hbm,
                    idxr_v, idxc_v, bufr_v, bufc_v, semr, semc, semo):
    cid = lax.axis_index("c")
    sid = lax.axis_index("s")
    wid = sid * _NC + cid
    pltpu.sync_copy(idxr_hbm.at[wid], idxr_v)
    pltpu.sync_copy(idxc_hbm.at[wid], idxc_v)
    pltpu.async_copy(trow_hbm.at[idxr_v.at[0]], bufr_v.at[0], semr)
    pltpu.async_copy(tcol_hbm.at[idxc_v.at[0]], bufc_v.at[0], semc)

    def chunk(j, carry):
        cur = lax.rem(j, 2)
        nxt = lax.rem(j + 1, 2)
        base = (wid * _NCH + j) * _CHUNK
        pltpu.make_async_copy(trow_hbm.at[idxr_v.at[j]],
                              bufr_v.at[cur], semr).wait()
        pltpu.make_async_copy(tcol_hbm.at[idxc_v.at[j]],
                              bufc_v.at[cur], semc).wait()

        @pl.when(j > 0)
        def _():
            pbase = (wid * _NCH + j - 1) * _CHUNK
            pltpu.make_async_copy(
                bufr_v.at[nxt],
                grow_hbm.at[pl.ds(pbase, _CHUNK), pl.ds(0, _GROW)],
                semo).wait()
            pltpu.make_async_copy(
                bufc_v.at[nxt],
                grow_hbm.at[pl.ds(pbase, _CHUNK), pl.ds(_GROW, _GCOL)],
                semo).wait()

        @pl.when(j + 1 < _NCH)
        def _():
            pltpu.async_copy(trow_hbm.at[idxr_v.at[j + 1]],
                             bufr_v.at[nxt], semr)
            pltpu.async_copy(tcol_hbm.at[idxc_v.at[j + 1]],
                             bufc_v.at[nxt], semc)

        pltpu.async_copy(
            bufr_v.at[cur],
            grow_hbm.at[pl.ds(base, _CHUNK), pl.ds(0, _GROW)], semo)
        pltpu.async_copy(
            bufc_v.at[cur],
            grow_hbm.at[pl.ds(base, _CHUNK), pl.ds(_GROW, _GCOL)], semo)
        return carry

    lax.fori_loop(0, _NCH, chunk, 0)
    lbase = (wid * _NCH + _NCH - 1) * _CHUNK
    lpar = (_NCH - 1) % 2
    pltpu.make_async_copy(
        bufr_v.at[lpar],
        grow_hbm.at[pl.ds(lbase, _CHUNK), pl.ds(0, _GROW)], semo).wait()
    pltpu.make_async_copy(
        bufc_v.at[lpar],
        grow_hbm.at[pl.ds(lbase, _CHUNK), pl.ds(_GROW, _GCOL)], semo).wait()


def _sc_gather(trow, tcol, idxr, idxc):
    mesh = plsc.VectorSubcoreMesh(core_axis_name="c", subcore_axis_name="s",
                                  num_cores=_NC, num_subcores=_NS)
    return pl.kernel(
        _sc_gather_body,
        out_type=jax.ShapeDtypeStruct((_EPAD, _GW), jnp.float32),
        mesh=mesh,
        scratch_types=[
            pltpu.VMEM((_NCH, _CHUNK), jnp.int32),
            pltpu.VMEM((_NCH, _CHUNK), jnp.int32),
            pltpu.VMEM((2, _CHUNK, _GROW), jnp.float32),
            pltpu.VMEM((2, _CHUNK, _GCOL), jnp.float32),
            pltpu.SemaphoreType.DMA,
            pltpu.SemaphoreType.DMA,
            pltpu.SemaphoreType.DMA,
        ],
        compiler_params=pltpu.CompilerParams(use_tc_tiling_on_sc=False),
    )(trow, tcol, idxr, idxc)


def _dot(a, b):
    return jnp.dot(a, b, preferred_element_type=jnp.float32)


def _tc_msg_body(g2_ref, w1_ref, w2_ref, r1_ref, s1_ref, r2_ref,
                 s2_ref, r3_ref, s3_ref, r4_ref, s4_ref, t4_ref, o3_ref,
                 o16_ref, out_ref):
    blk = g2_ref[...]
    g = jnp.concatenate([blk[:, 0:_GW], blk[:, _GW:2 * _GW]], axis=0)
    xs = g[:, 0:16]
    xv = g[:, 16:40]
    ev = g[:, 40:43] - g[:, 48:51]
    len2 = _dot(ev * ev, o3_ref[...])
    elen = jnp.sqrt(len2)
    d16 = _dot(elen, o16_ref[...])
    centers = lax.broadcasted_iota(jnp.int32, (1, _NUM_BASIS), 1).astype(
        jnp.float32) * np.float32(4.0 / (_NUM_BASIS - 1))
    rbf = jnp.exp(-8.0 * (d16 - centers) ** 2)
    h = jnp.maximum(_dot(rbf, w1_ref[...]), 0.0)
    w = _dot(h, w2_ref[...])

    rlen = jnp.maximum(elen, 1e-12)
    y1 = ev * (jnp.sqrt(3.0).astype(jnp.float32) / rlen)
    xvY = (xv[:, 0:8] * y1[:, 0:1] + xv[:, 8:16] * y1[:, 1:2]
           + xv[:, 16:24] * y1[:, 2:3])

    t = jnp.concatenate([xs, xvY], axis=1)
    out0 = _dot(w[:, 0:384] * _dot(t, r1_ref[...]), s1_ref[...])
    c1 = _dot(w[:, 384:512] * _dot(xs, r2_ref[...]), s2_ref[...])
    wd = w[:, 512:576]
    outs = [out0]
    for k in range(3):
        repd = _dot(g[:, 16 + 8 * k:24 + 8 * k], r3_ref[...])
        d1 = _dot(wd * repd, s3_ref[...])
        outs.append(c1 * y1[:, k:k + 1] + d1)
    msg = jnp.concatenate(outs, axis=1)
    half = blk.shape[0]
    z24 = jnp.zeros((half, 24), jnp.float32)
    out_ref[...] = jnp.concatenate(
        [msg[0:half], z24, msg[half:2 * half], z24], axis=1)


def _tp_consts():
    r1 = np.zeros((24, 384), np.float32)
    for j in range(256):
        r1[j // 16, j] = 1.0
    for j in range(128):
        r1[16 + j // 16, 256 + j] = 1.0
    s1 = np.zeros((384, 16), np.float32)
    for j in range(384):
        s1[j, j % 16] = 1.0
    r2 = np.zeros((16, 128), np.float32)
    for j in range(128):
        r2[j // 8, j] = 1.0
    s2 = np.zeros((128, 8), np.float32)
    for j in range(128):
        s2[j, j % 8] = 1.0
    r3 = np.zeros((8, 64), np.float32)
    for j in range(64):
        r3[j // 8, j] = 1.0
    s3 = np.zeros((64, 8), np.float32)
    for j in range(64):
        s3[j, j % 8] = 1.0
    r4 = np.zeros((3, 24), np.float32)
    for j in range(24):
        r4[j // 8, j] = 1.0
    s4 = np.zeros((24, 8), np.float32)
    for j in range(24):
        s4[j, j % 8] = 1.0
    t4 = np.zeros((8, 24), np.float32)
    for j in range(24):
        t4[j % 8, j] = 1.0
    o3 = np.ones((3, 1), np.float32)
    o16 = np.ones((1, 16), np.float32)
    mats = (r1, s1, r2, s2, r3, s3, r4, s4, t4, o3, o16)
    return tuple(jnp.asarray(m) for m in mats)


def _tc_msg(g2, w1p, w2p):
    grid = (_EPAD2 // _BE2,)
    consts = _tp_consts()
    full = lambda a: pl.BlockSpec(a.shape, lambda i: (0,) * a.ndim)
    return pl.pallas_call(
        _tc_msg_body,
        grid=grid,
        in_specs=[
            pl.BlockSpec((_BE2, 2 * _GW), lambda i: (i, 0)),
            pl.BlockSpec((_NUM_BASIS, _HIDDEN), lambda i: (0, 0)),
            pl.BlockSpec((_HIDDEN, _WN), lambda i: (0, 0)),
        ] + [full(c) for c in consts],
        out_specs=pl.BlockSpec((_BE2, 2 * _GW), lambda i: (i, 0)),
        out_shape=jax.ShapeDtypeStruct((_EPAD2, 2 * _GW), jnp.float32),
    )(g2, w1p, w2p, *consts)


def _sc_scatter_body(msg_hbm, idxd_hbm, zeros_hbm, out_hbm,
                     acc_shared, idx_v, buf_v, seml):
    cid = lax.axis_index("c")
    sid = lax.axis_index("s")
    wid = sid * _NC + cid
    pltpu.sync_copy(zeros_hbm, acc_shared.at[pl.ds(sid * _RPT, _RPT)])
    pltpu.sync_copy(idxd_hbm.at[wid], idx_v)
    pltpu.async_copy(msg_hbm.at[pl.ds(wid * _NCH * _CHUNK, _CHUNK)],
                     buf_v.at[0], seml)
    plsc.subcore_barrier()

    def chunk(j, carry):
        cur = lax.rem(j, 2)
        base = (wid * _NCH + j) * _CHUNK
        pltpu.make_async_copy(msg_hbm.at[pl.ds(base, _CHUNK)],
                              buf_v.at[cur], seml).wait()

        @pl.when(j + 1 < _NCH)
        def _():
            nbase = (wid * _NCH + j + 1) * _CHUNK
            pltpu.async_copy(msg_hbm.at[pl.ds(nbase, _CHUNK)],
                             buf_v.at[lax.rem(j + 1, 2)], seml)

        pltpu.sync_copy(buf_v.at[cur], acc_shared.at[idx_v.at[j]], add=True)
        return carry


    lax.fori_loop(0, _NCH, chunk, 0)
    plsc.subcore_barrier()
    pltpu.sync_copy(acc_shared.at[pl.ds(sid * _RPT, _RPT)],
                    out_hbm.at[cid].at[pl.ds(sid * _RPT, _RPT)])


def _sc_scatter(msg, idxd, zeros_blk):
    mesh = plsc.VectorSubcoreMesh(core_axis_name="c", subcore_axis_name="s",
                                  num_cores=_NC, num_subcores=_NS)
    return pl.kernel(
        _sc_scatter_body,
        out_type=jax.ShapeDtypeStruct((_NC, _NACC, _GW), jnp.float32),
        mesh=mesh,
        scratch_types=[
            pltpu.VMEM_SHARED((_NACC, _GW), jnp.float32),
            pltpu.VMEM((_NCH, _CHUNK), jnp.int32),
            pltpu.VMEM((2, _CHUNK, _GW), jnp.float32),
            pltpu.SemaphoreType.DMA,
        ],
        compiler_params=pltpu.CompilerParams(use_tc_tiling_on_sc=False),
    )(msg, idxd, zeros_blk)


def _tc_fin_body(p_ref, out_ref):
    s = p_ref[0] + p_ref[1]
    sc = s[:, 0:16]
    act = sc / (1.0 + jnp.exp(-sc))
    out_ref[...] = jnp.concatenate([act, s[:, 16:_GW]], axis=1)


def _tc_fin(partials):
    return pl.pallas_call(
        _tc_fin_body,
        out_shape=jax.ShapeDtypeStruct((_NACC, _GW), jnp.float32),
    )(partials)


def kernel(x, pos, edge_index, W1, W2):
    f32 = jnp.float32
    row = edge_index[0].astype(jnp.int32)
    col = edge_index[1].astype(jnp.int32)

    pw0 = np.sqrt(1.0 / 24.0)
    pw1 = np.sqrt(3.0 / 24.0)
    inv_s3 = 1.0 / np.sqrt(3.0)
    colscale = np.concatenate([
        np.full(256, pw0), np.full(128, pw0 * inv_s3),
        np.full(128, pw1 * inv_s3), np.full(64, pw1 * inv_s3),
    ]).astype(np.float32)
    w1p = W1 * np.float32(np.sqrt(2.0) / np.sqrt(_NUM_BASIS))
    w2p = (W2 * np.float32(1.0 / np.sqrt(_HIDDEN))) * colscale[None, :]

    xk = jnp.concatenate(
        [x[:, :16],
         x[:, 16:].reshape(_N, 8, 3).transpose(0, 2, 1).reshape(_N, 24)],
        axis=1)
    trow = jnp.concatenate([xk, pos, jnp.zeros((_N, 5), f32)], axis=1)
    tcol = jnp.concatenate([pos, jnp.zeros((_N, 13), f32)], axis=1)

    pad = _EPAD - _E
    idxr = jnp.pad(row, (0, pad)).reshape(_NW, _NCH, _CHUNK)
    idxc = jnp.pad(col, (0, pad)).reshape(_NW, _NCH, _CHUNK)
    idxd = jnp.pad(row, (0, pad), constant_values=_NACC - 1).reshape(
        _NW, _NCH, _CHUNK)
    zeros_blk = jnp.zeros((_RPT, _GW), f32)

    grow = _sc_gather(trow, tcol, idxr, idxc)
    g2 = grow.reshape(_EPAD2, 2 * _GW)
    msg2 = _tc_msg(g2, w1p, w2p)
    msg = msg2.reshape(_EPAD, _GW)
    partials = _sc_scatter(msg, idxd, zeros_blk)
    yfull = _tc_fin(partials)

    ys = yfull[:_N, 0:16]
    yv = yfull[:_N, 16:40].reshape(_N, 3, 8).transpose(0, 2, 1).reshape(_N, 24)
    return jnp.concatenate([ys, yv], axis=1)


# speedup vs baseline: 5.0838x; 1.0284x over previous
import functools
import numpy as np
import jax
import jax.numpy as jnp
from jax import lax
from jax.experimental import pallas as pl
from jax.experimental.pallas import tpu as pltpu
from jax.experimental.pallas import tpu_sc as plsc

_N = 10000
_E = 160000
_MUL0 = 16
_MUL1 = 8
_DIM = 40
_NUM_BASIS = 16
_HIDDEN = 64
_WN = 576

_NC = 2
_NS = 16
_NW = _NC * _NS
_CHUNK = 128
_NCH = 40
_EPAD = _NW * _NCH * _CHUNK
_EPAD2 = _EPAD // 2
_GROW = 48
_GCOL = 16
_GW = 64
_NACC = 10240
_RPT = _NACC // _NS

_BE2 = 1024


def _sc_gather_body(nch, trow_hbm, tcol_hbm, idxr_hbm, idxc_hbm,
                    grow_hbm, idxr_v, idxc_v, bufr_v, bufc_v,
                    semr, semc, semo):
    cid = lax.axis_index("c")
    sid = lax.axis_index("s")
    wid = sid * _NC + cid
    pltpu.sync_copy(idxr_hbm.at[wid], idxr_v)
    pltpu.sync_copy(idxc_hbm.at[wid], idxc_v)
    pltpu.async_copy(trow_hbm.at[idxr_v.at[0]], bufr_v.at[0], semr)
    pltpu.async_copy(tcol_hbm.at[idxc_v.at[0]], bufc_v.at[0], semc)

    def chunk(j, carry):
        cur = lax.rem(j, 2)
        nxt = lax.rem(j + 1, 2)
        base = (wid * nch + j) * _CHUNK
        pltpu.make_async_copy(trow_hbm.at[idxr_v.at[j]],
                              bufr_v.at[cur], semr).wait()
        pltpu.make_async_copy(tcol_hbm.at[idxc_v.at[j]],
                              bufc_v.at[cur], semc).wait()

        @pl.when(j > 0)
        def _():
            pbase = (wid * nch + j - 1) * _CHUNK
            pltpu.make_async_copy(
                bufr_v.at[nxt],
                grow_hbm.at[pl.ds(pbase, _CHUNK), pl.ds(0, _GROW)],
                semo).wait()
            pltpu.make_async_copy(
                bufc_v.at[nxt],
                grow_hbm.at[pl.ds(pbase, _CHUNK), pl.ds(_GROW, _GCOL)],
                semo).wait()

        @pl.when(j + 1 < nch)
        def _():
            pltpu.async_copy(trow_hbm.at[idxr_v.at[j + 1]],
                             bufr_v.at[nxt], semr)
            pltpu.async_copy(tcol_hbm.at[idxc_v.at[j + 1]],
                             bufc_v.at[nxt], semc)

        pltpu.async_copy(
            bufr_v.at[cur],
            grow_hbm.at[pl.ds(base, _CHUNK), pl.ds(0, _GROW)], semo)
        pltpu.async_copy(
            bufc_v.at[cur],
            grow_hbm.at[pl.ds(base, _CHUNK), pl.ds(_GROW, _GCOL)], semo)
        return carry

    lax.fori_loop(0, nch, chunk, 0)
    lbase = (wid * nch + nch - 1) * _CHUNK
    lpar = (nch - 1) % 2
    pltpu.make_async_copy(
        bufr_v.at[lpar],
        grow_hbm.at[pl.ds(lbase, _CHUNK), pl.ds(0, _GROW)], semo).wait()
    pltpu.make_async_copy(
        bufc_v.at[lpar],
        grow_hbm.at[pl.ds(lbase, _CHUNK), pl.ds(_GROW, _GCOL)], semo).wait()


def _sc_gather(trow, tcol, idxr, idxc):
    nch = idxr.shape[1]
    mesh = plsc.VectorSubcoreMesh(core_axis_name="c", subcore_axis_name="s",
                                  num_cores=_NC, num_subcores=_NS)
    return pl.kernel(
        functools.partial(_sc_gather_body, nch),
        out_type=jax.ShapeDtypeStruct((_NW * nch * _CHUNK, _GW), jnp.float32),
        mesh=mesh,
        scratch_types=[
            pltpu.VMEM((nch, _CHUNK), jnp.int32),
            pltpu.VMEM((nch, _CHUNK), jnp.int32),
            pltpu.VMEM((2, _CHUNK, _GROW), jnp.float32),
            pltpu.VMEM((2, _CHUNK, _GCOL), jnp.float32),
            pltpu.SemaphoreType.DMA,
            pltpu.SemaphoreType.DMA,
            pltpu.SemaphoreType.DMA,
        ],
        compiler_params=pltpu.CompilerParams(use_tc_tiling_on_sc=False),
    )(trow, tcol, idxr, idxc)


def _dot(a, b):
    return jnp.dot(a, b, preferred_element_type=jnp.float32)


def _tc_msg_body(g2_ref, w1_ref, w2_ref, r1_ref, s1_ref, r2_ref,
                 s2_ref, r3_ref, s3_ref, r4_ref, s4_ref, t4_ref, o3_ref,
                 o16_ref, out_ref):
    blk = g2_ref[...]
    g = jnp.concatenate([blk[:, 0:_GW], blk[:, _GW:2 * _GW]], axis=0)
    xs = g[:, 0:16]
    xv = g[:, 16:40]
    ev = g[:, 40:43] - g[:, 48:51]
    len2 = _dot(ev * ev, o3_ref[...])
    elen = jnp.sqrt(len2)
    d16 = _dot(elen, o16_ref[...])
    centers = lax.broadcasted_iota(jnp.int32, (1, _NUM_BASIS), 1).astype(
        jnp.float32) * np.float32(4.0 / (_NUM_BASIS - 1))
    rbf = jnp.exp(-8.0 * (d16 - centers) ** 2)
    h = jnp.maximum(_dot(rbf, w1_ref[...]), 0.0)
    w = _dot(h, w2_ref[...])

    rlen = jnp.maximum(elen, 1e-12)
    y1 = ev * (jnp.sqrt(3.0).astype(jnp.float32) / rlen)
    xvY = (xv[:, 0:8] * y1[:, 0:1] + xv[:, 8:16] * y1[:, 1:2]
           + xv[:, 16:24] * y1[:, 2:3])

    t = jnp.concatenate([xs, xvY], axis=1)
    out0 = _dot(w[:, 0:384] * _dot(t, r1_ref[...]), s1_ref[...])
    c1 = _dot(w[:, 384:512] * _dot(xs, r2_ref[...]), s2_ref[...])
    wd = w[:, 512:576]
    outs = [out0]
    for k in range(3):
        repd = _dot(g[:, 16 + 8 * k:24 + 8 * k], r3_ref[...])
        d1 = _dot(wd * repd, s3_ref[...])
        outs.append(c1 * y1[:, k:k + 1] + d1)
    msg = jnp.concatenate(outs, axis=1)
    half = blk.shape[0]
    z24 = jnp.zeros((half, 24), jnp.float32)
    out_ref[...] = jnp.concatenate(
        [msg[0:half], z24, msg[half:2 * half], z24], axis=1)


def _tp_consts():
    r1 = np.zeros((24, 384), np.float32)
    for j in range(256):
        r1[j // 16, j] = 1.0
    for j in range(128):
        r1[16 + j // 16, 256 + j] = 1.0
    s1 = np.zeros((384, 16), np.float32)
    for j in range(384):
        s1[j, j % 16] = 1.0
    r2 = np.zeros((16, 128), np.float32)
    for j in range(128):
        r2[j // 8, j] = 1.0
    s2 = np.zeros((128, 8), np.float32)
    for j in range(128):
        s2[j, j % 8] = 1.0
    r3 = np.zeros((8, 64), np.float32)
    for j in range(64):
        r3[j // 8, j] = 1.0
    s3 = np.zeros((64, 8), np.float32)
    for j in range(64):
        s3[j, j % 8] = 1.0
    r4 = np.zeros((3, 24), np.float32)
    for j in range(24):
        r4[j // 8, j] = 1.0
    s4 = np.zeros((24, 8), np.float32)
    for j in range(24):
        s4[j, j % 8] = 1.0
    t4 = np.zeros((8, 24), np.float32)
    for j in range(24):
        t4[j % 8, j] = 1.0
    o3 = np.ones((3, 1), np.float32)
    o16 = np.ones((1, 16), np.float32)
    mats = (r1, s1, r2, s2, r3, s3, r4, s4, t4, o3, o16)
    return tuple(jnp.asarray(m) for m in mats)


def _tc_msg(g2, w1p, w2p):
    grid = (g2.shape[0] // _BE2,)
    consts = _tp_consts()
    full = lambda a: pl.BlockSpec(a.shape, lambda i: (0,) * a.ndim)
    return pl.pallas_call(
        _tc_msg_body,
        grid=grid,
        in_specs=[
            pl.BlockSpec((_BE2, 2 * _GW), lambda i: (i, 0)),
            pl.BlockSpec((_NUM_BASIS, _HIDDEN), lambda i: (0, 0)),
            pl.BlockSpec((_HIDDEN, _WN), lambda i: (0, 0)),
        ] + [full(c) for c in consts],
        out_specs=pl.BlockSpec((_BE2, 2 * _GW), lambda i: (i, 0)),
        out_shape=jax.ShapeDtypeStruct(g2.shape, jnp.float32),
    )(g2, w1p, w2p, *consts)


def _sc_scatter_body(nch, msg_hbm, idxd_hbm, zeros_hbm, out_hbm,
                     acc_shared, idx_v, buf_v, seml):
    cid = lax.axis_index("c")
    sid = lax.axis_index("s")
    wid = sid * _NC + cid
    pltpu.sync_copy(zeros_hbm, acc_shared.at[pl.ds(sid * _RPT, _RPT)])
    pltpu.sync_copy(idxd_hbm.at[wid], idx_v)
    pltpu.async_copy(msg_hbm.at[pl.ds(wid * nch * _CHUNK, _CHUNK)],
                     buf_v.at[0], seml)
    plsc.subcore_barrier()

    def chunk(j, carry):
        cur = lax.rem(j, 2)
        base = (wid * nch + j) * _CHUNK
        pltpu.make_async_copy(msg_hbm.at[pl.ds(base, _CHUNK)],
                              buf_v.at[cur], seml).wait()

        @pl.when(j + 1 < nch)
        def _():
            nbase = (wid * nch + j + 1) * _CHUNK
            pltpu.async_copy(msg_hbm.at[pl.ds(nbase, _CHUNK)],
                             buf_v.at[lax.rem(j + 1, 2)], seml)

        pltpu.sync_copy(buf_v.at[cur], acc_shared.at[idx_v.at[j]], add=True)
        return carry


    lax.fori_loop(0, _NCH, chunk, 0)
    plsc.subcore_barrier()
    pltpu.sync_copy(acc_shared.at[pl.ds(sid * _RPT, _RPT)],
                    out_hbm.at[cid].at[pl.ds(sid * _RPT, _RPT)])


def _sc_scatter(msg, idxd, zeros_blk):
    nch = idxd.shape[1]
    mesh = plsc.VectorSubcoreMesh(core_axis_name="c", subcore_axis_name="s",
                                  num_cores=_NC, num_subcores=_NS)
    return pl.kernel(
        functools.partial(_sc_scatter_body, nch),
        out_type=jax.ShapeDtypeStruct((_NC, _NACC, _GW), jnp.float32),
        mesh=mesh,
        scratch_types=[
            pltpu.VMEM_SHARED((_NACC, _GW), jnp.float32),
            pltpu.VMEM((nch, _CHUNK), jnp.int32),
            pltpu.VMEM((2, _CHUNK, _GW), jnp.float32),
            pltpu.SemaphoreType.DMA,
        ],
        compiler_params=pltpu.CompilerParams(use_tc_tiling_on_sc=False),
    )(msg, idxd, zeros_blk)


def _tc_fin_body(p_ref, pm_ref, out_ref):
    s = p_ref[0] + p_ref[1]
    sc = s[:, 0:16]
    act = sc / (1.0 + jnp.exp(-sc))
    yv = _dot(s[:, 16:40], pm_ref[...])
    out_ref[...] = jnp.concatenate([act, yv], axis=1)


def _tc_fin(partials):
    pm = np.zeros((24, 24), np.float32)
    for k in range(3):
        for u in range(8):
            pm[8 * k + u, 3 * u + k] = 1.0
    return pl.pallas_call(
        _tc_fin_body,
        out_shape=jax.ShapeDtypeStruct((_NACC, _DIM), jnp.float32),
    )(partials, jnp.asarray(pm))


def kernel(x, pos, edge_index, W1, W2):
    f32 = jnp.float32
    row = edge_index[0].astype(jnp.int32)
    col = edge_index[1].astype(jnp.int32)

    pw0 = np.sqrt(1.0 / 24.0)
    pw1 = np.sqrt(3.0 / 24.0)
    inv_s3 = 1.0 / np.sqrt(3.0)
    colscale = np.concatenate([
        np.full(256, pw0), np.full(128, pw0 * inv_s3),
        np.full(128, pw1 * inv_s3), np.full(64, pw1 * inv_s3),
    ]).astype(np.float32)
    w1p = W1 * np.float32(np.sqrt(2.0) / np.sqrt(_NUM_BASIS))
    w2p = (W2 * np.float32(1.0 / np.sqrt(_HIDDEN))) * colscale[None, :]

    xk = jnp.concatenate(
        [x[:, :16],
         x[:, 16:].reshape(_N, 8, 3).transpose(0, 2, 1).reshape(_N, 24)],
        axis=1)
    trow = jnp.concatenate([xk, pos, jnp.zeros((_N, 5), f32)], axis=1)
    tcol = jnp.concatenate([pos, jnp.zeros((_N, 13), f32)], axis=1)

    pad = _EPAD - _E
    idxr = jnp.pad(row, (0, pad)).reshape(_NW, _NCH, _CHUNK)
    idxc = jnp.pad(col, (0, pad)).reshape(_NW, _NCH, _CHUNK)
    idxd = jnp.pad(row, (0, pad), constant_values=_NACC - 1).reshape(
        _NW, _NCH, _CHUNK)
    zeros_blk = jnp.zeros((_RPT, _GW), f32)

    grow = _sc_gather(trow, tcol, idxr, idxc)
    g2 = grow.reshape(_EPAD2, 2 * _GW)
    msg2 = _tc_msg(g2, w1p, w2p)
    msg = msg2.reshape(_EPAD, _GW)
    partials = _sc_scatter(msg, idxd, zeros_blk)
    yfull = _tc_fin(partials)
    return yfull[:_N]


# speedup vs baseline: 5.2230x; 1.0274x over previous
import functools
import numpy as np
import jax
import jax.numpy as jnp
from jax import lax
from jax.experimental import pallas as pl
from jax.experimental.pallas import tpu as pltpu
from jax.experimental.pallas import tpu_sc as plsc

_N = 10000
_E = 160000
_MUL0 = 16
_MUL1 = 8
_DIM = 40
_NUM_BASIS = 16
_HIDDEN = 64
_WN = 576

_NC = 2
_NS = 16
_NW = _NC * _NS
_CHUNK = 128
_NCH = 40
_EPAD = _NW * _NCH * _CHUNK
_EPAD2 = _EPAD // 2
_GROW = 48
_GCOL = 16
_GW = 64
_NACC = 10240
_RPT = _NACC // _NS

_BE2 = 1024


def _sc_gather_body(nch, trow_hbm, tcol_hbm, idxr_hbm, idxc_hbm,
                    grow_hbm, idxr_v, idxc_v, bufr_v, bufc_v,
                    semr, semc, semo):
    cid = lax.axis_index("c")
    sid = lax.axis_index("s")
    wid = sid * _NC + cid
    pltpu.sync_copy(idxr_hbm.at[wid], idxr_v)
    pltpu.sync_copy(idxc_hbm.at[wid], idxc_v)
    pltpu.async_copy(trow_hbm.at[idxr_v.at[0]], bufr_v.at[0], semr)
    pltpu.async_copy(tcol_hbm.at[idxc_v.at[0]], bufc_v.at[0], semc)
    pltpu.async_copy(trow_hbm.at[idxr_v.at[1]], bufr_v.at[1], semr)
    pltpu.async_copy(tcol_hbm.at[idxc_v.at[1]], bufc_v.at[1], semc)

    def outwait(jj):
        obase = (wid * nch + jj) * _CHUNK
        ob = lax.rem(jj, 3)
        pltpu.make_async_copy(
            bufr_v.at[ob],
            grow_hbm.at[pl.ds(obase, _CHUNK), pl.ds(0, _GROW)], semo).wait()
        pltpu.make_async_copy(
            bufc_v.at[ob],
            grow_hbm.at[pl.ds(obase, _CHUNK), pl.ds(_GROW, _GCOL)],
            semo).wait()

    def chunk(j, carry):
        cur = lax.rem(j, 3)
        base = (wid * nch + j) * _CHUNK

        @pl.when(j > 0)
        def _():
            outwait(j - 1)

        @pl.when(j + 2 < nch)
        def _():
            nx2 = lax.rem(j + 2, 3)
            pltpu.async_copy(trow_hbm.at[idxr_v.at[j + 2]],
                             bufr_v.at[nx2], semr)
            pltpu.async_copy(tcol_hbm.at[idxc_v.at[j + 2]],
                             bufc_v.at[nx2], semc)

        pltpu.make_async_copy(trow_hbm.at[idxr_v.at[j]],
                              bufr_v.at[cur], semr).wait()
        pltpu.make_async_copy(tcol_hbm.at[idxc_v.at[j]],
                              bufc_v.at[cur], semc).wait()
        pltpu.async_copy(
            bufr_v.at[cur],
            grow_hbm.at[pl.ds(base, _CHUNK), pl.ds(0, _GROW)], semo)
        pltpu.async_copy(
            bufc_v.at[cur],
            grow_hbm.at[pl.ds(base, _CHUNK), pl.ds(_GROW, _GCOL)], semo)
        return carry

    lax.fori_loop(0, nch, chunk, 0)
    outwait(nch - 1)


def _sc_gather(trow, tcol, idxr, idxc):
    nch = idxr.shape[1]
    mesh = plsc.VectorSubcoreMesh(core_axis_name="c", subcore_axis_name="s",
                                  num_cores=_NC, num_subcores=_NS)
    return pl.kernel(
        functools.partial(_sc_gather_body, nch),
        out_type=jax.ShapeDtypeStruct((_NW * nch * _CHUNK, _GW), jnp.float32),
        mesh=mesh,
        scratch_types=[
            pltpu.VMEM((nch, _CHUNK), jnp.int32),
            pltpu.VMEM((nch, _CHUNK), jnp.int32),
            pltpu.VMEM((3, _CHUNK, _GROW), jnp.float32),
            pltpu.VMEM((3, _CHUNK, _GCOL), jnp.float32),
            pltpu.SemaphoreType.DMA,
            pltpu.SemaphoreType.DMA,
            pltpu.SemaphoreType.DMA,
        ],
        compiler_params=pltpu.CompilerParams(use_tc_tiling_on_sc=False),
    )(trow, tcol, idxr, idxc)


def _dot(a, b):
    return jnp.dot(a, b, preferred_element_type=jnp.float32)


def _tc_msg_body(g2_ref, w1_ref, w2_ref, r1_ref, s1_ref, r2_ref,
                 s2_ref, r3_ref, s3_ref, r4_ref, s4_ref, t4_ref, o3_ref,
                 o16_ref, out_ref):
    blk = g2_ref[...]
    g = jnp.concatenate([blk[:, 0:_GW], blk[:, _GW:2 * _GW]], axis=0)
    xs = g[:, 0:16]
    xv = g[:, 16:40]
    ev = g[:, 40:43] - g[:, 48:51]
    len2 = _dot(ev * ev, o3_ref[...])
    elen = jnp.sqrt(len2)
    d16 = _dot(elen, o16_ref[...])
    centers = lax.broadcasted_iota(jnp.int32, (1, _NUM_BASIS), 1).astype(
        jnp.float32) * np.float32(4.0 / (_NUM_BASIS - 1))
    rbf = jnp.exp(-8.0 * (d16 - centers) ** 2)
    h = jnp.maximum(_dot(rbf, w1_ref[...]), 0.0)
    w = _dot(h, w2_ref[...])

    rlen = jnp.maximum(elen, 1e-12)
    y1 = ev * (jnp.sqrt(3.0).astype(jnp.float32) / rlen)
    xvY = (xv[:, 0:8] * y1[:, 0:1] + xv[:, 8:16] * y1[:, 1:2]
           + xv[:, 16:24] * y1[:, 2:3])

    t = jnp.concatenate([xs, xvY], axis=1)
    out0 = _dot(w[:, 0:384] * _dot(t, r1_ref[...]), s1_ref[...])
    c1 = _dot(w[:, 384:512] * _dot(xs, r2_ref[...]), s2_ref[...])
    wd = w[:, 512:576]
    outs = [out0]
    for k in range(3):
        repd = _dot(g[:, 16 + 8 * k:24 + 8 * k], r3_ref[...])
        d1 = _dot(wd * repd, s3_ref[...])
        outs.append(c1 * y1[:, k:k + 1] + d1)
    msg = jnp.concatenate(outs, axis=1)
    half = blk.shape[0]
    z24 = jnp.zeros((half, 24), jnp.float32)
    out_ref[...] = jnp.concatenate(
        [msg[0:half], z24, msg[half:2 * half], z24], axis=1)


def _tp_consts():
    r1 = np.zeros((24, 384), np.float32)
    for j in range(256):
        r1[j // 16, j] = 1.0
    for j in range(128):
        r1[16 + j // 16, 256 + j] = 1.0
    s1 = np.zeros((384, 16), np.float32)
    for j in range(384):
        s1[j, j % 16] = 1.0
    r2 = np.zeros((16, 128), np.float32)
    for j in range(128):
        r2[j // 8, j] = 1.0
    s2 = np.zeros((128, 8), np.float32)
    for j in range(128):
        s2[j, j % 8] = 1.0
    r3 = np.zeros((8, 64), np.float32)
    for j in range(64):
        r3[j // 8, j] = 1.0
    s3 = np.zeros((64, 8), np.float32)
    for j in range(64):
        s3[j, j % 8] = 1.0
    r4 = np.zeros((3, 24), np.float32)
    for j in range(24):
        r4[j // 8, j] = 1.0
    s4 = np.zeros((24, 8), np.float32)
    for j in range(24):
        s4[j, j % 8] = 1.0
    t4 = np.zeros((8, 24), np.float32)
    for j in range(24):
        t4[j % 8, j] = 1.0
    o3 = np.ones((3, 1), np.float32)
    o16 = np.ones((1, 16), np.float32)
    mats = (r1, s1, r2, s2, r3, s3, r4, s4, t4, o3, o16)
    return tuple(jnp.asarray(m) for m in mats)


def _tc_msg(g2, w1p, w2p):
    grid = (g2.shape[0] // _BE2,)
    consts = _tp_consts()
    full = lambda a: pl.BlockSpec(a.shape, lambda i: (0,) * a.ndim)
    return pl.pallas_call(
        _tc_msg_body,
        grid=grid,
        in_specs=[
            pl.BlockSpec((_BE2, 2 * _GW), lambda i: (i, 0)),
            pl.BlockSpec((_NUM_BASIS, _HIDDEN), lambda i: (0, 0)),
            pl.BlockSpec((_HIDDEN, _WN), lambda i: (0, 0)),
        ] + [full(c) for c in consts],
        out_specs=pl.BlockSpec((_BE2, 2 * _GW), lambda i: (i, 0)),
        out_shape=jax.ShapeDtypeStruct(g2.shape, jnp.float32),
    )(g2, w1p, w2p, *consts)


def _sc_scatter_body(nch, msg_hbm, idxd_hbm, zeros_hbm, out_hbm,
                     acc_shared, idx_v, buf_v, seml):
    cid = lax.axis_index("c")
    sid = lax.axis_index("s")
    wid = sid * _NC + cid
    pltpu.sync_copy(zeros_hbm, acc_shared.at[pl.ds(sid * _RPT, _RPT)])
    pltpu.sync_copy(idxd_hbm.at[wid], idx_v)
    pltpu.async_copy(msg_hbm.at[pl.ds(wid * nch * _CHUNK, _CHUNK)],
                     buf_v.at[0], seml)
    plsc.subcore_barrier()

    def chunk(j, carry):
        cur = lax.rem(j, 2)
        base = (wid * nch + j) * _CHUNK
        pltpu.make_async_copy(msg_hbm.at[pl.ds(base, _CHUNK)],
                              buf_v.at[cur], seml).wait()

        @pl.when(j + 1 < nch)
        def _():
            nbase = (wid * nch + j + 1) * _CHUNK
            pltpu.async_copy(msg_hbm.at[pl.ds(nbase, _CHUNK)],
                             buf_v.at[lax.rem(j + 1, 2)], seml)

        pltpu.sync_copy(buf_v.at[cur], acc_shared.at[idx_v.at[j]], add=True)
        return carry


    lax.fori_loop(0, _NCH, chunk, 0)
    plsc.subcore_barrier()
    pltpu.sync_copy(acc_shared.at[pl.ds(sid * _RPT, _RPT)],
                    out_hbm.at[cid].at[pl.ds(sid * _RPT, _RPT)])


def _sc_scatter(msg, idxd, zeros_blk):
    nch = idxd.shape[1]
    mesh = plsc.VectorSubcoreMesh(core_axis_name="c", subcore_axis_name="s",
                                  num_cores=_NC, num_subcores=_NS)
    return pl.kernel(
        functools.partial(_sc_scatter_body, nch),
        out_type=jax.ShapeDtypeStruct((_NC, _NACC, _GW), jnp.float32),
        mesh=mesh,
        scratch_types=[
            pltpu.VMEM_SHARED((_NACC, _GW), jnp.float32),
            pltpu.VMEM((nch, _CHUNK), jnp.int32),
            pltpu.VMEM((2, _CHUNK, _GW), jnp.float32),
            pltpu.SemaphoreType.DMA,
        ],
        compiler_params=pltpu.CompilerParams(use_tc_tiling_on_sc=False),
    )(msg, idxd, zeros_blk)


def _tc_fin_body(p_ref, pm_ref, out_ref):
    s = p_ref[0] + p_ref[1]
    sc = s[:, 0:16]
    act = sc / (1.0 + jnp.exp(-sc))
    yv = _dot(s[:, 16:40], pm_ref[...])
    out_ref[...] = jnp.concatenate([act, yv], axis=1)


def _tc_fin(partials):
    pm = np.zeros((24, 24), np.float32)
    for k in range(3):
        for u in range(8):
            pm[8 * k + u, 3 * u + k] = 1.0
    return pl.pallas_call(
        _tc_fin_body,
        out_shape=jax.ShapeDtypeStruct((_NACC, _DIM), jnp.float32),
    )(partials, jnp.asarray(pm))


def kernel(x, pos, edge_index, W1, W2):
    f32 = jnp.float32
    row = edge_index[0].astype(jnp.int32)
    col = edge_index[1].astype(jnp.int32)

    pw0 = np.sqrt(1.0 / 24.0)
    pw1 = np.sqrt(3.0 / 24.0)
    inv_s3 = 1.0 / np.sqrt(3.0)
    colscale = np.concatenate([
        np.full(256, pw0), np.full(128, pw0 * inv_s3),
        np.full(128, pw1 * inv_s3), np.full(64, pw1 * inv_s3),
    ]).astype(np.float32)
    w1p = W1 * np.float32(np.sqrt(2.0) / np.sqrt(_NUM_BASIS))
    w2p = (W2 * np.float32(1.0 / np.sqrt(_HIDDEN))) * colscale[None, :]

    xk = jnp.concatenate(
        [x[:, :16],
         x[:, 16:].reshape(_N, 8, 3).transpose(0, 2, 1).reshape(_N, 24)],
        axis=1)
    trow = jnp.concatenate([xk, pos, jnp.zeros((_N, 5), f32)], axis=1)
    trow = jnp.concatenate([trow, jnp.zeros((8, _GROW), f32)], axis=0)
    tcol = jnp.concatenate([pos, jnp.zeros((_N, 13), f32)], axis=1)

    pad = _EPAD - _E
    idxr = jnp.pad(row, (0, pad), constant_values=_N).reshape(
        _NW, _NCH, _CHUNK)
    idxc = jnp.pad(col, (0, pad)).reshape(_NW, _NCH, _CHUNK)
    zeros_blk = jnp.zeros((_RPT, _GW), f32)

    grow = _sc_gather(trow, tcol, idxr, idxc)
    g2 = grow.reshape(_EPAD2, 2 * _GW)
    msg2 = _tc_msg(g2, w1p, w2p)
    msg = msg2.reshape(_EPAD, _GW)
    partials = _sc_scatter(msg, idxr, zeros_blk)
    yfull = _tc_fin(partials)
    return yfull[:_N]
